# baseline (speedup 1.0000x reference)
import jax
import jax.numpy as jnp
from jax import lax
from jax.experimental import pallas as pl
from jax.experimental.pallas import tpu as pltpu
from jax.experimental.pallas import tpu_sc as plsc

N_NODES = 10000
N_EDGES = 320000
D_FEAT = 128
D_EDGE = 16
D_HID = 128

_NC = 2
_NS = 16
_NW = _NC * _NS
_EPW = N_EDGES // _NW
_G = 40
_NBUF = 4
_NCHUNK = _EPW // _G


def _proj_body(node_ref, ws_ref, wr_ref, ps_ref, pr_ref):
    x = node_ref[...]
    ps_ref[...] = jnp.dot(x, ws_ref[...], preferred_element_type=jnp.float32)
    pr_ref[...] = jnp.dot(x, wr_ref[...], preferred_element_type=jnp.float32)


def _edge_body(e_ref, we_ref, b_ref, t_ref):
    t_ref[...] = (
        jnp.dot(e_ref[...], we_ref[...], preferred_element_type=jnp.float32)
        + b_ref[...]
    )


def _sc_body(ps_hbm, pr_hbm, t_hbm, si_hbm, ri_hbm, out_hbm,
             si_all, ri_all, rs_v, rr_v, tt_v, ob_v, *sems):
    gsem_s = sems[0:_NBUF]
    gsem_r = sems[_NBUF:2 * _NBUF]
    gsem_t = sems[2 * _NBUF:3 * _NBUF]
    osem = sems[3 * _NBUF:4 * _NBUF]
    sem_i = sems[4 * _NBUF]
    wid = lax.axis_index("s") * _NC + lax.axis_index("c")
    base = wid * _EPW

    ci1 = pltpu.async_copy(si_hbm.at[pl.ds(base, _EPW)], si_all, sem_i)
    ci2 = pltpu.async_copy(ri_hbm.at[pl.ds(base, _EPW)], ri_all, sem_i)
    ci1.wait()
    ci2.wait()

    def issue(c, b):
        loc = c * _G
        g_s = pltpu.async_copy(
            ps_hbm.at[si_all.at[pl.ds(loc, _G)]], rs_v.at[b], gsem_s[b])
        g_r = pltpu.async_copy(
            pr_hbm.at[ri_all.at[pl.ds(loc, _G)]], rr_v.at[b], gsem_r[b])
        g_t = pltpu.async_copy(
            t_hbm.at[pl.ds(base + loc, _G)], tt_v.at[b], gsem_t[b])
        del g_s, g_r, g_t

    def drain(b):
        pltpu.make_async_copy(
            t_hbm.at[pl.ds(0, _G)], rs_v.at[b], gsem_s[b]).wait()
        pltpu.make_async_copy(
            t_hbm.at[pl.ds(0, _G)], rr_v.at[b], gsem_r[b]).wait()
        pltpu.make_async_copy(
            t_hbm.at[pl.ds(0, _G)], tt_v.at[b], gsem_t[b]).wait()

    def wait_store(b):
        pltpu.make_async_copy(
            ob_v.at[b], out_hbm.at[pl.ds(0, _G)], osem[b]).wait()

    def compute(b):
        def row(i, carry):
            for k in range(D_HID // 16):
                sl = pl.ds(k * 16, 16)
                v = rs_v[b, i, sl] + rr_v[b, i, sl] + tt_v[b, i, sl]
                ob_v[b, i, sl] = jnp.maximum(v, 0.0)
            return carry
        lax.fori_loop(0, _G, row, 0, unroll=2)

    def process(c, b):
        drain(b)

        @pl.when(c >= _NBUF)
        def _():
            wait_store(b)

        compute(b)
        st = pltpu.async_copy(
            ob_v.at[b], out_hbm.at[pl.ds(base + c * _G, _G)], osem[b])
        del st

        @pl.when(c + _NBUF < _NCHUNK)
        def _():
            issue(c + _NBUF, b)

    for b in range(_NBUF):
        issue(b, b)

    def group(g, carry):
        for b in range(_NBUF):
            process(g * _NBUF + b, b)
        return carry

    lax.fori_loop(0, _NCHUNK // _NBUF, group, 0)
    for b in range(_NCHUNK % _NBUF):
        process((_NCHUNK // _NBUF) * _NBUF + b, b)
    for b in range(_NBUF):
        wait_store(b)


def kernel(node_attr, edge_attr, edge_index, W, b):
    senders = edge_index[0].astype(jnp.int32)
    receivers = edge_index[1].astype(jnp.int32)
    w_s = W[:D_FEAT]
    w_r = W[D_FEAT:2 * D_FEAT]
    w_e = W[2 * D_FEAT:]
    b2 = b.reshape(1, D_HID)

    bm = 2000
    p_s, p_r = pl.pallas_call(
        _proj_body,
        grid=(N_NODES // bm,),
        in_specs=[
            pl.BlockSpec((bm, D_FEAT), lambda i: (i, 0)),
            pl.BlockSpec((D_FEAT, D_HID), lambda i: (0, 0)),
            pl.BlockSpec((D_FEAT, D_HID), lambda i: (0, 0)),
        ],
        out_specs=[
            pl.BlockSpec((bm, D_HID), lambda i: (i, 0)),
            pl.BlockSpec((bm, D_HID), lambda i: (i, 0)),
        ],
        out_shape=[
            jax.ShapeDtypeStruct((N_NODES, D_HID), jnp.float32),
            jax.ShapeDtypeStruct((N_NODES, D_HID), jnp.float32),
        ],
    )(node_attr, w_s, w_r)

    bme = 6400
    t_edge = pl.pallas_call(
        _edge_body,
        grid=(N_EDGES // bme,),
        in_specs=[
            pl.BlockSpec((bme, D_EDGE), lambda i: (i, 0)),
            pl.BlockSpec((D_EDGE, D_HID), lambda i: (0, 0)),
            pl.BlockSpec((1, D_HID), lambda i: (0, 0)),
        ],
        out_specs=pl.BlockSpec((bme, D_HID), lambda i: (i, 0)),
        out_shape=jax.ShapeDtypeStruct((N_EDGES, D_HID), jnp.float32),
    )(edge_attr, w_e, b2)

    sc_call = pl.kernel(
        _sc_body,
        out_type=jax.ShapeDtypeStruct((N_EDGES, D_HID), jnp.float32),
        mesh=plsc.VectorSubcoreMesh(core_axis_name="c", subcore_axis_name="s"),
        compiler_params=pltpu.CompilerParams(use_tc_tiling_on_sc=False),
        scratch_types=[
            pltpu.VMEM((_EPW,), jnp.int32),
            pltpu.VMEM((_EPW,), jnp.int32),
            pltpu.VMEM((_NBUF, _G, D_HID), jnp.float32),
            pltpu.VMEM((_NBUF, _G, D_HID), jnp.float32),
            pltpu.VMEM((_NBUF, _G, D_HID), jnp.float32),
            pltpu.VMEM((_NBUF, _G, D_HID), jnp.float32),
        ] + [pltpu.SemaphoreType.DMA] * (4 * _NBUF + 1),
    )
    return sc_call(p_s, p_r, t_edge, senders, receivers)


# speedup vs baseline: 1.8635x; 1.8635x over previous
import jax
import jax.numpy as jnp
import numpy as np
from jax import lax
from jax.experimental import pallas as pl
from jax.experimental.pallas import tpu as pltpu
from jax.experimental.pallas import tpu_sc as plsc

N_NODES = 10000
N_EDGES = 320000
D_FEAT = 128
D_EDGE = 16
D_HID = 128
_H = N_EDGES // 2
_DW = D_HID // 2

_NC = 2
_NS = 16
_NW = _NC * _NS
_RPW = _H // _NW
_G = 40
_NBUF = 3
_NCHUNK = _RPW // _G

_MASK_HI = np.uint32(0xFFFF0000)


def _pack_halves(lo_f32, hi_f32):
    lo = lax.bitcast_convert_type(
        lo_f32.astype(jnp.bfloat16), jnp.uint16).astype(jnp.uint32)
    hi = lax.bitcast_convert_type(
        hi_f32.astype(jnp.bfloat16), jnp.uint16).astype(jnp.uint32)
    return lax.bitcast_convert_type(lo | (hi << 16), jnp.int32)


def _proj_body(node_ref, ws_ref, wr_ref, ps_ref, pr_ref):
    x = node_ref[...]
    ps = jnp.dot(x, ws_ref[...], preferred_element_type=jnp.float32)
    pr = jnp.dot(x, wr_ref[...], preferred_element_type=jnp.float32)
    ps_ref[...] = _pack_halves(ps[:, :_DW], ps[:, _DW:])
    pr_ref[...] = _pack_halves(pr[:, :_DW], pr[:, _DW:])


def _edge_body(e1_ref, e2_ref, we_ref, b_ref, t_ref):
    t1 = jnp.dot(e1_ref[...], we_ref[...],
                 preferred_element_type=jnp.float32) + b_ref[...]
    t2 = jnp.dot(e2_ref[...], we_ref[...],
                 preferred_element_type=jnp.float32) + b_ref[...]
    t_ref[...] = _pack_halves(t1, t2)


def _sc_body(ps_hbm, pr_hbm, t_hbm, si_hbm, ri_hbm, out_hbm,
             si_all, ri_all, sl_v, sh_v, rl_v, rh_v, tt_v, ol_v, oh_v,
             *sems):
    gs = [sems[b * 7:(b + 1) * 7] for b in range(_NBUF)]
    sem_i = sems[7 * _NBUF]
    wid = lax.axis_index("s") * _NC + lax.axis_index("c")
    rbase = wid * _RPW

    ca = pltpu.async_copy(si_hbm.at[pl.ds(rbase, _RPW)],
                          si_all.at[pl.ds(0, _RPW)], sem_i)
    cb = pltpu.async_copy(si_hbm.at[pl.ds(_H + rbase, _RPW)],
                          si_all.at[pl.ds(_RPW, _RPW)], sem_i)
    cc = pltpu.async_copy(ri_hbm.at[pl.ds(rbase, _RPW)],
                          ri_all.at[pl.ds(0, _RPW)], sem_i)
    cd = pltpu.async_copy(ri_hbm.at[pl.ds(_H + rbase, _RPW)],
                          ri_all.at[pl.ds(_RPW, _RPW)], sem_i)
    ca.wait()
    cb.wait()
    cc.wait()
    cd.wait()

    def issue(c, b):
        loc = c * _G
        s1 = pltpu.async_copy(
            ps_hbm.at[si_all.at[pl.ds(loc, _G)]], sl_v.at[b], gs[b][0])
        s2 = pltpu.async_copy(
            ps_hbm.at[si_all.at[pl.ds(_RPW + loc, _G)]], sh_v.at[b], gs[b][1])
        s3 = pltpu.async_copy(
            pr_hbm.at[ri_all.at[pl.ds(loc, _G)]], rl_v.at[b], gs[b][2])
        s4 = pltpu.async_copy(
            pr_hbm.at[ri_all.at[pl.ds(_RPW + loc, _G)]], rh_v.at[b], gs[b][3])
        s5 = pltpu.async_copy(
            t_hbm.at[pl.ds(rbase + loc, _G)], tt_v.at[b], gs[b][4])
        del s1, s2, s3, s4, s5

    def drain(b):
        for j, dst in ((0, sl_v), (1, sh_v), (2, rl_v), (3, rh_v)):
            pltpu.make_async_copy(
                ps_hbm.at[pl.ds(0, _G)], dst.at[b], gs[b][j]).wait()
        pltpu.make_async_copy(
            t_hbm.at[pl.ds(0, _G)], tt_v.at[b], gs[b][4]).wait()

    def wait_stores(b):
        pltpu.make_async_copy(
            ol_v.at[b], out_hbm.at[pl.ds(0, _G)], gs[b][5]).wait()
        pltpu.make_async_copy(
            oh_v.at[b], out_hbm.at[pl.ds(0, _G)], gs[b][6]).wait()

    def as_f32(x):
        return lax.bitcast_convert_type(x, jnp.float32)

    def as_u32(x):
        return lax.bitcast_convert_type(x, jnp.uint32)

    def lo_part(w):
        return as_f32(jnp.left_shift(w, 16))

    def hi_part(w):
        return as_f32(w & _MASK_HI)

    def compute(b):
        def row(i, carry):
            for k in range(_DW // 16):
                slf = pl.ds(k * 16, 16)
                slb = pl.ds(_DW + k * 16, 16)
                w_sl = as_u32(sl_v[b, i, slf])
                w_sh = as_u32(sh_v[b, i, slf])
                w_rl = as_u32(rl_v[b, i, slf])
                w_rh = as_u32(rh_v[b, i, slf])
                t_f = as_u32(tt_v[b, i, slf])
                t_b = as_u32(tt_v[b, i, slb])
                ol_v[b, i, slf] = jnp.maximum(
                    lo_part(w_sl) + lo_part(w_rl) + lo_part(t_f), 0.0)
                ol_v[b, i, slb] = jnp.maximum(
                    hi_part(w_sl) + hi_part(w_rl) + lo_part(t_b), 0.0)
                oh_v[b, i, slf] = jnp.maximum(
                    lo_part(w_sh) + lo_part(w_rh) + hi_part(t_f), 0.0)
                oh_v[b, i, slb] = jnp.maximum(
                    hi_part(w_sh) + hi_part(w_rh) + hi_part(t_b), 0.0)
            return carry
        lax.fori_loop(0, _G, row, 0, unroll=2)

    def process(c, b):
        drain(b)

        @pl.when(c >= _NBUF)
        def _():
            wait_stores(b)

        compute(b)
        loc = c * _G
        st1 = pltpu.async_copy(
            ol_v.at[b], out_hbm.at[pl.ds(rbase + loc, _G)], gs[b][5])
        st2 = pltpu.async_copy(
            oh_v.at[b], out_hbm.at[pl.ds(_H + rbase + loc, _G)], gs[b][6])
        del st1, st2

        @pl.when(c + _NBUF < _NCHUNK)
        def _():
            issue(c + _NBUF, b)

    for b in range(_NBUF):
        issue(b, b)

    def group(g, carry):
        for b in range(_NBUF):
            process(g * _NBUF + b, b)
        return carry

    lax.fori_loop(0, _NCHUNK // _NBUF, group, 0)
    for b in range(_NCHUNK % _NBUF):
        process((_NCHUNK // _NBUF) * _NBUF + b, b)
    for b in range(_NBUF):
        wait_stores(b)


def kernel(node_attr, edge_attr, edge_index, W, b):
    senders = edge_index[0].astype(jnp.int32)
    receivers = edge_index[1].astype(jnp.int32)
    w_s = W[:D_FEAT]
    w_r = W[D_FEAT:2 * D_FEAT]
    w_e = W[2 * D_FEAT:]
    b2 = b.reshape(1, D_HID)

    bm = 2000
    ps_w, pr_w = pl.pallas_call(
        _proj_body,
        grid=(N_NODES // bm,),
        in_specs=[
            pl.BlockSpec((bm, D_FEAT), lambda i: (i, 0)),
            pl.BlockSpec((D_FEAT, D_HID), lambda i: (0, 0)),
            pl.BlockSpec((D_FEAT, D_HID), lambda i: (0, 0)),
        ],
        out_specs=[
            pl.BlockSpec((bm, _DW), lambda i: (i, 0)),
            pl.BlockSpec((bm, _DW), lambda i: (i, 0)),
        ],
        out_shape=[
            jax.ShapeDtypeStruct((N_NODES, _DW), jnp.int32),
            jax.ShapeDtypeStruct((N_NODES, _DW), jnp.int32),
        ],
    )(node_attr, w_s, w_r)

    bme = 6400
    nsteps = _H // bme
    t_w = pl.pallas_call(
        _edge_body,
        grid=(nsteps,),
        in_specs=[
            pl.BlockSpec((bme, D_EDGE), lambda i: (i, 0)),
            pl.BlockSpec((bme, D_EDGE), lambda i: (i + nsteps, 0)),
            pl.BlockSpec((D_EDGE, D_HID), lambda i: (0, 0)),
            pl.BlockSpec((1, D_HID), lambda i: (0, 0)),
        ],
        out_specs=pl.BlockSpec((bme, D_HID), lambda i: (i, 0)),
        out_shape=jax.ShapeDtypeStruct((_H, D_HID), jnp.int32),
    )(edge_attr, edge_attr, w_e, b2)

    sc_call = pl.kernel(
        _sc_body,
        out_type=jax.ShapeDtypeStruct((N_EDGES, D_HID), jnp.float32),
        mesh=plsc.VectorSubcoreMesh(core_axis_name="c", subcore_axis_name="s"),
        compiler_params=pltpu.CompilerParams(use_tc_tiling_on_sc=False),
        scratch_types=[
            pltpu.VMEM((2 * _RPW,), jnp.int32),
            pltpu.VMEM((2 * _RPW,), jnp.int32),
            pltpu.VMEM((_NBUF, _G, _DW), jnp.int32),
            pltpu.VMEM((_NBUF, _G, _DW), jnp.int32),
            pltpu.VMEM((_NBUF, _G, _DW), jnp.int32),
            pltpu.VMEM((_NBUF, _G, _DW), jnp.int32),
            pltpu.VMEM((_NBUF, _G, D_HID), jnp.int32),
            pltpu.VMEM((_NBUF, _G, D_HID), jnp.float32),
            pltpu.VMEM((_NBUF, _G, D_HID), jnp.float32),
        ] + [pltpu.SemaphoreType.DMA] * (7 * _NBUF + 1),
    )
    return sc_call(ps_w, pr_w, t_w, senders, receivers)


# speedup vs baseline: 1.8668x; 1.0018x over previous
import jax
import jax.numpy as jnp
import numpy as np
from jax import lax
from jax.experimental import pallas as pl
from jax.experimental.pallas import tpu as pltpu
from jax.experimental.pallas import tpu_sc as plsc

N_NODES = 10000
N_EDGES = 320000
D_FEAT = 128
D_EDGE = 16
D_HID = 128
_H = N_EDGES // 2
_DW = D_HID // 2

_NC = 2
_NS = 16
_NW = _NC * _NS
_RPW = _H // _NW
_G = 40
_NBUF = 3
_NCHUNK = _RPW // _G

_MASK_HI = np.uint32(0xFFFF0000)


def _pack_halves(lo_f32, hi_f32):
    lo = lax.bitcast_convert_type(
        lo_f32.astype(jnp.bfloat16), jnp.uint16).astype(jnp.uint32)
    hi = lax.bitcast_convert_type(
        hi_f32.astype(jnp.bfloat16), jnp.uint16).astype(jnp.uint32)
    return lax.bitcast_convert_type(lo | (hi << 16), jnp.int32)


def _tc_body(node_ref, e1_ref, e2_ref, ws_ref, wr_ref, we_ref, b_ref,
             ps_ref, pr_ref, t_ref):
    x = node_ref[...]
    ps = jnp.dot(x, ws_ref[...], preferred_element_type=jnp.float32)
    pr = jnp.dot(x, wr_ref[...], preferred_element_type=jnp.float32)
    ps_ref[...] = _pack_halves(ps[:, :_DW], ps[:, _DW:])
    pr_ref[...] = _pack_halves(pr[:, :_DW], pr[:, _DW:])
    t1 = jnp.dot(e1_ref[...], we_ref[...],
                 preferred_element_type=jnp.float32) + b_ref[...]
    t2 = jnp.dot(e2_ref[...], we_ref[...],
                 preferred_element_type=jnp.float32) + b_ref[...]
    t_ref[...] = _pack_halves(t1, t2)


def _sc_body(ps_hbm, pr_hbm, t_hbm, si_hbm, ri_hbm, out_hbm,
             si_all, ri_all, sl_v, sh_v, rl_v, rh_v, tt_v, ol_v, oh_v,
             *sems):
    gs = [sems[b * 7:(b + 1) * 7] for b in range(_NBUF)]
    sem_i = sems[7 * _NBUF]
    wid = lax.axis_index("s") * _NC + lax.axis_index("c")
    rbase = wid * _RPW

    ca = pltpu.async_copy(si_hbm.at[pl.ds(rbase, _RPW)],
                          si_all.at[pl.ds(0, _RPW)], sem_i)
    cb = pltpu.async_copy(si_hbm.at[pl.ds(_H + rbase, _RPW)],
                          si_all.at[pl.ds(_RPW, _RPW)], sem_i)
    cc = pltpu.async_copy(ri_hbm.at[pl.ds(rbase, _RPW)],
                          ri_all.at[pl.ds(0, _RPW)], sem_i)
    cd = pltpu.async_copy(ri_hbm.at[pl.ds(_H + rbase, _RPW)],
                          ri_all.at[pl.ds(_RPW, _RPW)], sem_i)
    ca.wait()
    cb.wait()
    cc.wait()
    cd.wait()

    def issue(c, b):
        loc = c * _G
        s1 = pltpu.async_copy(
            ps_hbm.at[si_all.at[pl.ds(loc, _G)]], sl_v.at[b], gs[b][0])
        s2 = pltpu.async_copy(
            ps_hbm.at[si_all.at[pl.ds(_RPW + loc, _G)]], sh_v.at[b], gs[b][1])
        s3 = pltpu.async_copy(
            pr_hbm.at[ri_all.at[pl.ds(loc, _G)]], rl_v.at[b], gs[b][2])
        s4 = pltpu.async_copy(
            pr_hbm.at[ri_all.at[pl.ds(_RPW + loc, _G)]], rh_v.at[b], gs[b][3])
        s5 = pltpu.async_copy(
            t_hbm.at[pl.ds(rbase + loc, _G)], tt_v.at[b], gs[b][4])
        del s1, s2, s3, s4, s5

    def drain(b):
        for j, dst in ((0, sl_v), (1, sh_v), (2, rl_v), (3, rh_v)):
            pltpu.make_async_copy(
                ps_hbm.at[pl.ds(0, _G)], dst.at[b], gs[b][j]).wait()
        pltpu.make_async_copy(
            t_hbm.at[pl.ds(0, _G)], tt_v.at[b], gs[b][4]).wait()

    def wait_stores(b):
        pltpu.make_async_copy(
            ol_v.at[b], out_hbm.at[pl.ds(0, _G)], gs[b][5]).wait()
        pltpu.make_async_copy(
            oh_v.at[b], out_hbm.at[pl.ds(0, _G)], gs[b][6]).wait()

    def as_f32(x):
        return lax.bitcast_convert_type(x, jnp.float32)

    def as_u32(x):
        return lax.bitcast_convert_type(x, jnp.uint32)

    def lo_part(w):
        return as_f32(jnp.left_shift(w, 16))

    def hi_part(w):
        return as_f32(w & _MASK_HI)

    def compute(b):
        def row(i, carry):
            for k in range(_DW // 16):
                slf = pl.ds(k * 16, 16)
                slb = pl.ds(_DW + k * 16, 16)
                w_sl = as_u32(sl_v[b, i, slf])
                w_sh = as_u32(sh_v[b, i, slf])
                w_rl = as_u32(rl_v[b, i, slf])
                w_rh = as_u32(rh_v[b, i, slf])
                t_f = as_u32(tt_v[b, i, slf])
                t_b = as_u32(tt_v[b, i, slb])
                ol_v[b, i, slf] = jnp.maximum(
                    lo_part(w_sl) + lo_part(w_rl) + lo_part(t_f), 0.0)
                ol_v[b, i, slb] = jnp.maximum(
                    hi_part(w_sl) + hi_part(w_rl) + lo_part(t_b), 0.0)
                oh_v[b, i, slf] = jnp.maximum(
                    lo_part(w_sh) + lo_part(w_rh) + hi_part(t_f), 0.0)
                oh_v[b, i, slb] = jnp.maximum(
                    hi_part(w_sh) + hi_part(w_rh) + hi_part(t_b), 0.0)
            return carry
        lax.fori_loop(0, _G, row, 0, unroll=2)

    def process(c, b):
        drain(b)

        @pl.when(c >= _NBUF)
        def _():
            wait_stores(b)

        compute(b)
        loc = c * _G
        st1 = pltpu.async_copy(
            ol_v.at[b], out_hbm.at[pl.ds(rbase + loc, _G)], gs[b][5])
        st2 = pltpu.async_copy(
            oh_v.at[b], out_hbm.at[pl.ds(_H + rbase + loc, _G)], gs[b][6])
        del st1, st2

        @pl.when(c + _NBUF < _NCHUNK)
        def _():
            issue(c + _NBUF, b)

    for b in range(_NBUF):
        issue(b, b)

    def group(g, carry):
        for b in range(_NBUF):
            process(g * _NBUF + b, b)
        return carry

    lax.fori_loop(0, _NCHUNK // _NBUF, group, 0)
    for b in range(_NCHUNK % _NBUF):
        process((_NCHUNK // _NBUF) * _NBUF + b, b)
    for b in range(_NBUF):
        wait_stores(b)


def kernel(node_attr, edge_attr, edge_index, W, b):
    senders = edge_index[0].astype(jnp.int32)
    receivers = edge_index[1].astype(jnp.int32)
    w_s = W[:D_FEAT]
    w_r = W[D_FEAT:2 * D_FEAT]
    w_e = W[2 * D_FEAT:]
    b2 = b.reshape(1, D_HID)

    bme = 6400
    nsteps = _H // bme
    bm = N_NODES // nsteps
    ps_w, pr_w, t_w = pl.pallas_call(
        _tc_body,
        grid=(nsteps,),
        in_specs=[
            pl.BlockSpec((bm, D_FEAT), lambda i: (i, 0)),
            pl.BlockSpec((bme, D_EDGE), lambda i: (i, 0)),
            pl.BlockSpec((bme, D_EDGE), lambda i: (i + nsteps, 0)),
            pl.BlockSpec((D_FEAT, D_HID), lambda i: (0, 0)),
            pl.BlockSpec((D_FEAT, D_HID), lambda i: (0, 0)),
            pl.BlockSpec((D_EDGE, D_HID), lambda i: (0, 0)),
            pl.BlockSpec((1, D_HID), lambda i: (0, 0)),
        ],
        out_specs=[
            pl.BlockSpec((bm, _DW), lambda i: (i, 0)),
            pl.BlockSpec((bm, _DW), lambda i: (i, 0)),
            pl.BlockSpec((bme, D_HID), lambda i: (i, 0)),
        ],
        out_shape=[
            jax.ShapeDtypeStruct((N_NODES, _DW), jnp.int32),
            jax.ShapeDtypeStruct((N_NODES, _DW), jnp.int32),
            jax.ShapeDtypeStruct((_H, D_HID), jnp.int32),
        ],
    )(node_attr, edge_attr, edge_attr, w_s, w_r, w_e, b2)

    sc_call = pl.kernel(
        _sc_body,
        out_type=jax.ShapeDtypeStruct((N_EDGES, D_HID), jnp.float32),
        mesh=plsc.VectorSubcoreMesh(core_axis_name="c", subcore_axis_name="s"),
        compiler_params=pltpu.CompilerParams(use_tc_tiling_on_sc=False),
        scratch_types=[
            pltpu.VMEM((2 * _RPW,), jnp.int32),
            pltpu.VMEM((2 * _RPW,), jnp.int32),
            pltpu.VMEM((_NBUF, _G, _DW), jnp.int32),
            pltpu.VMEM((_NBUF, _G, _DW), jnp.int32),
            pltpu.VMEM((_NBUF, _G, _DW), jnp.int32),
            pltpu.VMEM((_NBUF, _G, _DW), jnp.int32),
            pltpu.VMEM((_NBUF, _G, D_HID), jnp.int32),
            pltpu.VMEM((_NBUF, _G, D_HID), jnp.float32),
            pltpu.VMEM((_NBUF, _G, D_HID), jnp.float32),
        ] + [pltpu.SemaphoreType.DMA] * (7 * _NBUF + 1),
    )
    return sc_call(ps_w, pr_w, t_w, senders, receivers)


# speedup vs baseline: 1.8714x; 1.0024x over previous
import jax
import jax.numpy as jnp
import numpy as np
from jax import lax
from jax.experimental import pallas as pl
from jax.experimental.pallas import tpu as pltpu
from jax.experimental.pallas import tpu_sc as plsc

N_NODES = 10000
N_EDGES = 320000
D_FEAT = 128
D_EDGE = 16
D_HID = 128
_H = N_EDGES // 2
_DW = D_HID // 2

_NC = 2
_NS = 16
_NW = _NC * _NS
_RPW = _H // _NW
_G = 40
_NBUF = 3
_NCHUNK = _RPW // _G

_MASK_HI = np.uint32(0xFFFF0000)


def _pack_halves(lo_f32, hi_f32):
    lo = jnp.right_shift(
        lax.bitcast_convert_type(lo_f32, jnp.uint32) + np.uint32(0x8000), 16)
    hi = (lax.bitcast_convert_type(hi_f32, jnp.uint32)
          + np.uint32(0x8000)) & _MASK_HI
    return lax.bitcast_convert_type(lo | hi, jnp.int32)


def _tc_body(node_ref, e1_ref, e2_ref, ws_ref, wr_ref, we_ref, b_ref,
             ps_ref, pr_ref, t_ref):
    x = node_ref[...]
    ps = jnp.dot(x, ws_ref[...], preferred_element_type=jnp.float32)
    pr = jnp.dot(x, wr_ref[...], preferred_element_type=jnp.float32)
    ps_ref[...] = _pack_halves(ps[:, :_DW], ps[:, _DW:])
    pr_ref[...] = _pack_halves(pr[:, :_DW], pr[:, _DW:])
    t1 = jnp.dot(e1_ref[...], we_ref[...],
                 preferred_element_type=jnp.float32) + b_ref[...]
    t2 = jnp.dot(e2_ref[...], we_ref[...],
                 preferred_element_type=jnp.float32) + b_ref[...]
    t_ref[...] = _pack_halves(t1, t2)


def _sc_body(ps_hbm, pr_hbm, t_hbm, si_hbm, ri_hbm, out_hbm,
             si_all, ri_all, sl_v, sh_v, rl_v, rh_v, tt_v, ol_v, oh_v,
             *sems):
    gs = [sems[b * 7:(b + 1) * 7] for b in range(_NBUF)]
    sem_i = sems[7 * _NBUF]
    wid = lax.axis_index("s") * _NC + lax.axis_index("c")
    rbase = wid * _RPW

    ca = pltpu.async_copy(si_hbm.at[pl.ds(rbase, _RPW)],
                          si_all.at[pl.ds(0, _RPW)], sem_i)
    cb = pltpu.async_copy(si_hbm.at[pl.ds(_H + rbase, _RPW)],
                          si_all.at[pl.ds(_RPW, _RPW)], sem_i)
    cc = pltpu.async_copy(ri_hbm.at[pl.ds(rbase, _RPW)],
                          ri_all.at[pl.ds(0, _RPW)], sem_i)
    cd = pltpu.async_copy(ri_hbm.at[pl.ds(_H + rbase, _RPW)],
                          ri_all.at[pl.ds(_RPW, _RPW)], sem_i)
    ca.wait()
    cb.wait()
    cc.wait()
    cd.wait()

    def issue(c, b):
        loc = c * _G
        s1 = pltpu.async_copy(
            ps_hbm.at[si_all.at[pl.ds(loc, _G)]], sl_v.at[b], gs[b][0])
        s2 = pltpu.async_copy(
            ps_hbm.at[si_all.at[pl.ds(_RPW + loc, _G)]], sh_v.at[b], gs[b][1])
        s3 = pltpu.async_copy(
            pr_hbm.at[ri_all.at[pl.ds(loc, _G)]], rl_v.at[b], gs[b][2])
        s4 = pltpu.async_copy(
            pr_hbm.at[ri_all.at[pl.ds(_RPW + loc, _G)]], rh_v.at[b], gs[b][3])
        s5 = pltpu.async_copy(
            t_hbm.at[pl.ds(rbase + loc, _G)], tt_v.at[b], gs[b][4])
        del s1, s2, s3, s4, s5

    def drain(b):
        for j, dst in ((0, sl_v), (1, sh_v), (2, rl_v), (3, rh_v)):
            pltpu.make_async_copy(
                ps_hbm.at[pl.ds(0, _G)], dst.at[b], gs[b][j]).wait()
        pltpu.make_async_copy(
            t_hbm.at[pl.ds(0, _G)], tt_v.at[b], gs[b][4]).wait()

    def wait_stores(b):
        pltpu.make_async_copy(
            ol_v.at[b], out_hbm.at[pl.ds(0, _G)], gs[b][5]).wait()
        pltpu.make_async_copy(
            oh_v.at[b], out_hbm.at[pl.ds(0, _G)], gs[b][6]).wait()

    def as_f32(x):
        return lax.bitcast_convert_type(x, jnp.float32)

    def as_u32(x):
        return lax.bitcast_convert_type(x, jnp.uint32)

    def lo_part(w):
        return as_f32(jnp.left_shift(w, 16))

    def hi_part(w):
        return as_f32(w & _MASK_HI)

    def compute(b):
        def row(i, carry):
            for k in range(_DW // 16):
                slf = pl.ds(k * 16, 16)
                slb = pl.ds(_DW + k * 16, 16)
                w_sl = as_u32(sl_v[b, i, slf])
                w_sh = as_u32(sh_v[b, i, slf])
                w_rl = as_u32(rl_v[b, i, slf])
                w_rh = as_u32(rh_v[b, i, slf])
                t_f = as_u32(tt_v[b, i, slf])
                t_b = as_u32(tt_v[b, i, slb])
                ol_v[b, i, slf] = jnp.maximum(
                    lo_part(w_sl) + lo_part(w_rl) + lo_part(t_f), 0.0)
                ol_v[b, i, slb] = jnp.maximum(
                    hi_part(w_sl) + hi_part(w_rl) + lo_part(t_b), 0.0)
                oh_v[b, i, slf] = jnp.maximum(
                    lo_part(w_sh) + lo_part(w_rh) + hi_part(t_f), 0.0)
                oh_v[b, i, slb] = jnp.maximum(
                    hi_part(w_sh) + hi_part(w_rh) + hi_part(t_b), 0.0)
            return carry
        lax.fori_loop(0, _G, row, 0, unroll=2)

    def process(c, b):
        drain(b)

        @pl.when(c >= _NBUF)
        def _():
            wait_stores(b)

        compute(b)
        loc = c * _G
        st1 = pltpu.async_copy(
            ol_v.at[b], out_hbm.at[pl.ds(rbase + loc, _G)], gs[b][5])
        st2 = pltpu.async_copy(
            oh_v.at[b], out_hbm.at[pl.ds(_H + rbase + loc, _G)], gs[b][6])
        del st1, st2

        @pl.when(c + _NBUF < _NCHUNK)
        def _():
            issue(c + _NBUF, b)

    for b in range(_NBUF):
        issue(b, b)

    def group(g, carry):
        for b in range(_NBUF):
            process(g * _NBUF + b, b)
        return carry

    lax.fori_loop(0, _NCHUNK // _NBUF, group, 0)
    for b in range(_NCHUNK % _NBUF):
        process((_NCHUNK // _NBUF) * _NBUF + b, b)
    for b in range(_NBUF):
        wait_stores(b)


def kernel(node_attr, edge_attr, edge_index, W, b):
    senders = edge_index[0].astype(jnp.int32)
    receivers = edge_index[1].astype(jnp.int32)
    w_s = W[:D_FEAT]
    w_r = W[D_FEAT:2 * D_FEAT]
    w_e = W[2 * D_FEAT:]
    b2 = b.reshape(1, D_HID)

    bme = 6400
    nsteps = _H // bme
    bm = N_NODES // nsteps
    ps_w, pr_w, t_w = pl.pallas_call(
        _tc_body,
        grid=(nsteps,),
        in_specs=[
            pl.BlockSpec((bm, D_FEAT), lambda i: (i, 0)),
            pl.BlockSpec((bme, D_EDGE), lambda i: (i, 0)),
            pl.BlockSpec((bme, D_EDGE), lambda i: (i + nsteps, 0)),
            pl.BlockSpec((D_FEAT, D_HID), lambda i: (0, 0)),
            pl.BlockSpec((D_FEAT, D_HID), lambda i: (0, 0)),
            pl.BlockSpec((D_EDGE, D_HID), lambda i: (0, 0)),
            pl.BlockSpec((1, D_HID), lambda i: (0, 0)),
        ],
        out_specs=[
            pl.BlockSpec((bm, _DW), lambda i: (i, 0)),
            pl.BlockSpec((bm, _DW), lambda i: (i, 0)),
            pl.BlockSpec((bme, D_HID), lambda i: (i, 0)),
        ],
        out_shape=[
            jax.ShapeDtypeStruct((N_NODES, _DW), jnp.int32),
            jax.ShapeDtypeStruct((N_NODES, _DW), jnp.int32),
            jax.ShapeDtypeStruct((_H, D_HID), jnp.int32),
        ],
    )(node_attr, edge_attr, edge_attr, w_s, w_r, w_e, b2)

    sc_call = pl.kernel(
        _sc_body,
        out_type=jax.ShapeDtypeStruct((N_EDGES, D_HID), jnp.float32),
        mesh=plsc.VectorSubcoreMesh(core_axis_name="c", subcore_axis_name="s"),
        compiler_params=pltpu.CompilerParams(use_tc_tiling_on_sc=False),
        scratch_types=[
            pltpu.VMEM((2 * _RPW,), jnp.int32),
            pltpu.VMEM((2 * _RPW,), jnp.int32),
            pltpu.VMEM((_NBUF, _G, _DW), jnp.int32),
            pltpu.VMEM((_NBUF, _G, _DW), jnp.int32),
            pltpu.VMEM((_NBUF, _G, _DW), jnp.int32),
            pltpu.VMEM((_NBUF, _G, _DW), jnp.int32),
            pltpu.VMEM((_NBUF, _G, D_HID), jnp.int32),
            pltpu.VMEM((_NBUF, _G, D_HID), jnp.float32),
            pltpu.VMEM((_NBUF, _G, D_HID), jnp.float32),
        ] + [pltpu.SemaphoreType.DMA] * (7 * _NBUF + 1),
    )
    return sc_call(ps_w, pr_w, t_w, senders, receivers)
